# Initial kernel scaffold; baseline (speedup 1.0000x reference)
import jax
import jax.numpy as jnp
from jax import lax
from jax.experimental import pallas as pl
from jax.experimental.pallas import tpu as pltpu


def _probe_fps(px_ref, qx_ref):
    px = px_ref[...]
    Bq, N = px.shape
    lane = lax.broadcasted_iota(jnp.int32, (Bq, N), 1)

    def step(i, carry):
        mind, lpx = carry
        d = (px - lpx) * (px - lpx)
        mind = jnp.minimum(mind, d)
        m = jnp.max(mind, axis=1, keepdims=True)
        cand = jnp.where(mind == m, lane, N)
        j = jnp.min(cand, axis=1, keepdims=True)
        sel = lane == j
        lpx = jnp.sum(jnp.where(sel, px, 0.0), axis=1, keepdims=True)
        qx_ref[:, pl.ds(i, 1)] = lpx
        return (mind, lpx)

    qx_ref[:, pl.ds(0, 1)] = px[:, 0:1]
    lax.fori_loop(1, 16, step, (jnp.full((Bq, N), 1e30, jnp.float32), px[:, 0:1]))


def _probe_reshape(x_ref, o_ref):
    x = x_ref[...]
    o_ref[...] = jnp.max(x.reshape(128, 64, 64), axis=1)


def _probe_topk(x_ref, ov_ref, oi_ref):
    x = x_ref[...]
    v, i = lax.top_k(x, 64)
    ov_ref[...] = v
    oi_ref[...] = i


def _probe_dots(g_ref, w_ref, w2_ref, o_ref):
    g = g_ref[...]
    w = w_ref[...]
    h = jax.nn.relu(jnp.dot(g, w, preferred_element_type=jnp.float32))
    h = jnp.dot(h, w2_ref[...], preferred_element_type=jnp.float32)
    o_ref[...] = h


def _probe_strided(x_ref, o_ref):
    x = x_ref[...]
    for _ in range(6):
        x = jnp.maximum(x[0::2], x[1::2])
    o_ref[...] = x


def kernel(pos, params, batch):
    px = pos[: 8 * 2048, 0].reshape(8, 2048)
    qx = pl.pallas_call(
        _probe_fps,
        out_shape=jax.ShapeDtypeStruct((8, 1024), jnp.float32),
    )(px)

    x = jnp.broadcast_to(pos[:64, 0].reshape(1, 64), (8192, 64))
    r1 = pl.pallas_call(
        _probe_reshape,
        out_shape=jax.ShapeDtypeStruct((128, 64), jnp.float32),
    )(x)

    t = pos[: 256 * 64, 0].reshape(256, 64)
    t = jnp.concatenate([t] * 32, axis=1)
    tv, ti = pl.pallas_call(
        _probe_topk,
        out_shape=(
            jax.ShapeDtypeStruct((256, 64), jnp.float32),
            jax.ShapeDtypeStruct((256, 64), jnp.int32),
        ),
    )(t)

    g = jnp.pad(pos[:8192, :3], ((0, 0), (0, 1)))
    w = jnp.pad(params["sa1_w1"].T, ((0, 0), (0, 0)))
    w = jnp.pad(params["sa1_w1"].T, ((0, 1), (0, 0)))
    d1 = pl.pallas_call(
        _probe_dots,
        out_shape=jax.ShapeDtypeStruct((8192, 64), jnp.float32),
    )(g, w, params["sa1_w2"])

    s1 = pl.pallas_call(
        _probe_strided,
        out_shape=jax.ShapeDtypeStruct((128, 64), jnp.float32),
    )(x)

    return qx, r1, tv, ti, d1, s1


# speedup vs baseline: 1.0983x; 1.0983x over previous
import functools

import jax
import jax.numpy as jnp
from jax import lax
from jax.experimental import pallas as pl
from jax.experimental.pallas import tpu as pltpu

B, P = 8, 2048
N1, N2, N3 = 1024, 256, 64
KN = 64
G = 2500

INTERPRET = False


def _fps_level(px, py, pz, n_sample):
    Bc, N = px.shape
    lane = lax.broadcasted_iota(jnp.int32, (Bc, N), 1)
    qlane = lax.broadcasted_iota(jnp.int32, (Bc, n_sample), 1)

    def step(i, carry):
        mind, lpx, lpy, lpz, qx, qy, qz = carry
        dx = px - lpx
        dy = py - lpy
        dz = pz - lpz
        d = (dx * dx + dy * dy) + dz * dz
        mind = jnp.minimum(mind, d)
        m = jnp.max(mind, axis=1, keepdims=True)
        cand = jnp.where(mind == m, lane, N)
        j = jnp.min(cand, axis=1, keepdims=True)
        sel = lane == j
        lpx = jnp.sum(jnp.where(sel, px, 0.0), axis=1, keepdims=True)
        lpy = jnp.sum(jnp.where(sel, py, 0.0), axis=1, keepdims=True)
        lpz = jnp.sum(jnp.where(sel, pz, 0.0), axis=1, keepdims=True)
        qx = jnp.where(qlane == i, lpx, qx)
        qy = jnp.where(qlane == i, lpy, qy)
        qz = jnp.where(qlane == i, lpz, qz)
        return (mind, lpx, lpy, lpz, qx, qy, qz)

    zer = jnp.zeros((Bc, n_sample), jnp.float32)
    init = (
        jnp.full((Bc, N), 1e30, jnp.float32),
        px[:, 0:1],
        py[:, 0:1],
        pz[:, 0:1],
        jnp.where(qlane == 0, px[:, 0:1], zer),
        jnp.where(qlane == 0, py[:, 0:1], zer),
        jnp.where(qlane == 0, pz[:, 0:1], zer),
    )
    out = lax.fori_loop(1, n_sample, step, init)
    return out[4], out[5], out[6]


def _fps_kernel(px_ref, py_ref, pz_ref,
                q1x_ref, q1y_ref, q1z_ref,
                q2x_ref, q2y_ref, q2z_ref,
                q3x_ref, q3y_ref, q3z_ref):
    px, py, pz = px_ref[...], py_ref[...], pz_ref[...]
    q1x, q1y, q1z = _fps_level(px, py, pz, N1)
    q1x_ref[...], q1y_ref[...], q1z_ref[...] = q1x, q1y, q1z
    q2x, q2y, q2z = _fps_level(q1x, q1y, q1z, N2)
    q2x_ref[...], q2y_ref[...], q2z_ref[...] = q2x, q2y, q2z
    q3x, q3y, q3z = _fps_level(q2x, q2y, q2z, N3)
    q3x_ref[...], q3y_ref[...], q3z_ref[...] = q3x, q3y, q3z


def _run_fps(px, py, pz):
    sh = lambda n: jax.ShapeDtypeStruct((B, n), jnp.float32)
    return pl.pallas_call(
        _fps_kernel,
        out_shape=(sh(N1), sh(N1), sh(N1), sh(N2), sh(N2), sh(N2),
                   sh(N3), sh(N3), sh(N3)),
        interpret=INTERPRET,
    )(px, py, pz)


def _mlp_body(rel_ref, xg_ref, cnt_ref, w1p_ref, w1x_ref, w2_ref, b1_ref,
              b2_ref, out_ref, *, qb, C):
    rel = rel_ref[0]
    h = jnp.dot(rel, w1p_ref[...], preferred_element_type=jnp.float32)
    if xg_ref is not None:
        h = h + jnp.dot(xg_ref[0], w1x_ref[...],
                        preferred_element_type=jnp.float32)
    h = jax.nn.relu(h + b1_ref[...])
    h = jnp.dot(h, w2_ref[...], preferred_element_type=jnp.float32)
    h = h + b2_ref[...]
    h3 = h.reshape(qb, KN, C)
    kidx = lax.broadcasted_iota(jnp.int32, (qb, KN, C), 1)
    h3 = jnp.where(kidx < cnt_ref[0], h3, -1e9)
    grp = jnp.max(h3, axis=1)
    out_ref[0] = jnp.where(grp <= -1e8, 0.0, grp)


def _run_mlp(rel, xg, cnt, w1p, w1x, w2t, b1, b2, Q, C, qb):
    nq = Q // qb
    pairs = qb * KN
    Cin = 0 if xg is None else xg.shape[-1]

    specs = [pl.BlockSpec((1, pairs, 4), lambda b, q: (b, q, 0))]
    args = [rel]
    if xg is not None:
        specs.append(pl.BlockSpec((1, pairs, Cin), lambda b, q: (b, q, 0)))
        args.append(xg)
    specs.append(pl.BlockSpec((1, qb, 1, 1), lambda b, q: (b, q, 0, 0)))
    args.append(cnt)
    wfull = lambda a: pl.BlockSpec(a.shape, lambda b, q: (0,) * a.ndim)
    for a in (w1p,) + (() if xg is None else (w1x,)) + (w2t, b1, b2):
        specs.append(wfull(a))
        args.append(a)

    body = functools.partial(_mlp_body, qb=qb, C=C)
    if xg is None:
        body2 = lambda rel_ref, cnt_ref, w1p_ref, w2_ref, b1_ref, b2_ref, out_ref: body(
            rel_ref, None, cnt_ref, w1p_ref, None, w2_ref, b1_ref, b2_ref, out_ref)
    else:
        body2 = body

    return pl.pallas_call(
        body2,
        grid=(B, nq),
        in_specs=specs,
        out_specs=pl.BlockSpec((1, qb, C), lambda b, q: (b, q, 0)),
        out_shape=jax.ShapeDtypeStruct((B, Q, C), jnp.float32),
        interpret=INTERPRET,
    )(*args)


def _final_body(xg_ref, rel_ref, cnt_ref, posT_ref, eps_ref, grid_ref,
                w3p_ref, w3x_ref, w3t_ref, b31_ref, b32_ref,
                muw_ref, mub_ref, lvw_ref, lvb_ref,
                f1wz_ref, f1wg_ref, f1b1_ref, f1w2_ref, f1b2_ref, f1w3_ref,
                f1b3_ref,
                f2wz_ref, f2wp_ref, f2b1_ref, f2w2_ref, f2b2_ref, f2w3_ref,
                f2b3_ref,
                mu_ref, recon_ref, loss_ref, ch_ref, kl_ref, acc_ref):
    b = pl.program_id(0)
    f32 = jnp.float32

    h = jnp.dot(rel_ref[0], w3p_ref[...], preferred_element_type=f32)
    h = h + jnp.dot(xg_ref[0], w3x_ref[...], preferred_element_type=f32)
    h = jax.nn.relu(h + b31_ref[...])
    h = jnp.dot(h, w3t_ref[...], preferred_element_type=f32) + b32_ref[...]
    h3 = h.reshape(N3, KN, 256)
    kidx = lax.broadcasted_iota(jnp.int32, (N3, KN, 256), 1)
    h3 = jnp.where(kidx < cnt_ref[0], h3, -1e9)
    x3 = jnp.max(h3, axis=1)
    x3 = jnp.where(x3 <= -1e8, 0.0, x3)
    pooled = jnp.max(x3, axis=0, keepdims=True)

    mu = jnp.dot(pooled, muw_ref[...], preferred_element_type=f32) + mub_ref[...]
    lv = jnp.dot(pooled, lvw_ref[...], preferred_element_type=f32) + lvb_ref[...]
    z = mu + jnp.exp(0.5 * lv) * eps_ref[0]

    gr = grid_ref[...]
    zt = jnp.dot(z, f1wz_ref[...], preferred_element_type=f32)
    h1 = jax.nn.relu(
        jnp.dot(gr, f1wg_ref[...], preferred_element_type=f32) + zt
        + f1b1_ref[...])
    h1 = jax.nn.relu(
        jnp.dot(h1, f1w2_ref[...], preferred_element_type=f32) + f1b2_ref[...])
    x1g = jnp.dot(h1, f1w3_ref[...], preferred_element_type=f32) + f1b3_ref[...]

    zt2 = jnp.dot(z, f2wz_ref[...], preferred_element_type=f32)
    h2 = jax.nn.relu(
        jnp.dot(x1g, f2wp_ref[...], preferred_element_type=f32) + zt2
        + f2b1_ref[...])
    h2 = jax.nn.relu(
        jnp.dot(h2, f2w2_ref[...], preferred_element_type=f32) + f2b2_ref[...])
    recon = jnp.dot(h2, f2w3_ref[...], preferred_element_type=f32) + f2b3_ref[...]

    mu_ref[0] = mu
    recon_ref[0] = recon

    posT = posT_ref[0]
    tsq = jnp.sum(posT * posT, axis=0, keepdims=True)
    psq = jnp.sum(recon * recon, axis=1, keepdims=True)
    cross = jnp.dot(recon, posT, preferred_element_type=f32)
    d2 = psq + tsq - 2.0 * cross
    mA = jnp.min(d2, axis=1)
    mB = jnp.min(d2, axis=0)
    dA = jnp.sqrt(jnp.maximum(mA, 0.0) + 1e-12)
    dB = jnp.sqrt(jnp.maximum(mB, 0.0) + 1e-12)
    ch_part = jnp.sum(dA) / G + jnp.sum(dB) / P

    kl_part = jnp.sum(1.0 + lv - mu * mu - jnp.exp(lv))

    @pl.when(b == 0)
    def _():
        acc_ref[0] = ch_part
        acc_ref[1] = kl_part

    @pl.when(b > 0)
    def _():
        acc_ref[0] += ch_part
        acc_ref[1] += kl_part

    @pl.when(b == B - 1)
    def _():
        ch = acc_ref[0] / B
        kl = -0.5 * acc_ref[1] / B
        ch_ref[0, 0] = ch
        kl_ref[0, 0] = kl
        loss_ref[0, 0] = ch + 0.001 * kl


def _run_final(xg3, rel3, cnt3, posT, eps, gridc, wd):
    wfull = lambda a: pl.BlockSpec(a.shape, lambda b: (0,) * a.ndim)
    specs = [
        pl.BlockSpec((1, N3 * KN, 128), lambda b: (b, 0, 0)),
        pl.BlockSpec((1, N3 * KN, 4), lambda b: (b, 0, 0)),
        pl.BlockSpec((1, N3, 1, 1), lambda b: (b, 0, 0, 0)),
        pl.BlockSpec((1, 3, P), lambda b: (b, 0, 0)),
        pl.BlockSpec((1, 1, 64), lambda b: (b, 0, 0)),
        wfull(gridc),
    ]
    args = [xg3, rel3, cnt3, posT, eps, gridc]
    for a in wd:
        specs.append(wfull(a))
        args.append(a)
    sm = pltpu.SMEM
    out_shape = (
        jax.ShapeDtypeStruct((B, 1, 64), jnp.float32),
        jax.ShapeDtypeStruct((B, G, 3), jnp.float32),
        jax.ShapeDtypeStruct((1, 1), jnp.float32),
        jax.ShapeDtypeStruct((1, 1), jnp.float32),
        jax.ShapeDtypeStruct((1, 1), jnp.float32),
    )
    out_specs = (
        pl.BlockSpec((1, 1, 64), lambda b: (b, 0, 0)),
        pl.BlockSpec((1, G, 3), lambda b: (b, 0, 0)),
        pl.BlockSpec(memory_space=sm),
        pl.BlockSpec(memory_space=sm),
        pl.BlockSpec(memory_space=sm),
    )
    return pl.pallas_call(
        _final_body,
        grid=(B,),
        in_specs=specs,
        out_specs=out_specs,
        out_shape=out_shape,
        scratch_shapes=[pltpu.SMEM((2,), jnp.float32)],
        interpret=INTERPRET,
    )(*args)


def _select_gather(qx, qy, qz, px, py, pz, r, feats=None):
    N = px.shape[-1]
    Q = qx.shape[-1]

    def one(qx1, qy1, qz1, px1, py1, pz1, f1):
        dx = qx1[:, None] - px1[None, :]
        dy = qy1[:, None] - py1[None, :]
        dz = qz1[:, None] - pz1[None, :]
        d2 = (dx * dx + dy * dy) + dz * dz
        mask = d2 <= r * r
        iota = jnp.arange(N, dtype=jnp.int32)
        selv = jnp.where(mask, -iota, -N - 1)
        vals, _ = lax.top_k(selv, KN)
        got = vals > -N - 1
        nbr = jnp.where(got, -vals, 0)
        cnt = jnp.sum(mask.astype(jnp.int32), axis=1)
        cnt = jnp.minimum(cnt, KN)
        relx = px1[nbr] - qx1[:, None]
        rely = py1[nbr] - qy1[:, None]
        relz = pz1[nbr] - qz1[:, None]
        rel = jnp.stack(
            [relx, rely, relz, jnp.zeros_like(relx)], axis=-1
        ).reshape(Q * KN, 4)
        fg = None if f1 is None else f1[nbr].reshape(Q * KN, -1)
        return rel, cnt.reshape(Q, 1, 1), fg

    return jax.vmap(one)(qx, qy, qz, px, py, pz, feats)


def _tw(w):
    return jnp.asarray(w.T, jnp.float32)


def _padt(w, k=4):
    wt = w.T
    return jnp.pad(wt, ((0, k - wt.shape[0]), (0, 0)))


def kernel(pos, params, batch):
    pr = pos.reshape(B, P, 3)
    px = pr[:, :, 0]
    py = pr[:, :, 1]
    pz = pr[:, :, 2]

    (q1x, q1y, q1z, q2x, q2y, q2z, q3x, q3y, q3z) = _run_fps(px, py, pz)

    p = params
    rel1, cnt1, _ = _select_gather(q1x, q1y, q1z, px, py, pz, 0.2)
    x1 = _run_mlp(rel1, None, cnt1, _padt(p["sa1_w1"]), None,
                  _tw(p["sa1_w2"]), p["sa1_b1"][None], p["sa1_b2"][None],
                  N1, 64, 128)

    rel2, cnt2, xg2 = _select_gather(q2x, q2y, q2z, q1x, q1y, q1z, 0.4,
                                     feats=x1)
    x2 = _run_mlp(rel2, xg2, cnt2, _padt(p["sa2_w1"][:, 64:]),
                  _tw(p["sa2_w1"][:, :64]), _tw(p["sa2_w2"]),
                  p["sa2_b1"][None], p["sa2_b2"][None], N2, 128, 128)

    rel3, cnt3, xg3 = _select_gather(q3x, q3y, q3z, q2x, q2y, q2z, 0.8,
                                     feats=x2)

    eps = jax.random.normal(jax.random.key(42), (B, 64), dtype=jnp.float32)
    eps = eps.reshape(B, 1, 64)
    xs = jnp.linspace(-0.3, 0.3, 50)
    gx, gy = jnp.meshgrid(xs, xs, indexing="ij")
    gridc = jnp.stack([gx.ravel(), gy.ravel()], axis=-1).astype(jnp.float32)

    posT = jnp.stack([px, py, pz], axis=1)

    wd = [
        _padt(p["sa3_w1"][:, 128:]), _tw(p["sa3_w1"][:, :128]),
        _tw(p["sa3_w2"]), p["sa3_b1"][None], p["sa3_b2"][None],
        _tw(p["mu_w"]), p["mu_b"][None], _tw(p["lv_w"]), p["lv_b"][None],
        _tw(p["f1_w1"][:, :64]), _tw(p["f1_w1"][:, 64:]), p["f1_b1"][None],
        _tw(p["f1_w2"]), p["f1_b2"][None], _tw(p["f1_w3"]), p["f1_b3"][None],
        _tw(p["f2_w1"][:, :64]), _tw(p["f2_w1"][:, 64:]), p["f2_b1"][None],
        _tw(p["f2_w2"]), p["f2_b2"][None], _tw(p["f2_w3"]), p["f2_b3"][None],
    ]
    mu, recon, loss, ch, kl = _run_final(xg3, rel3, cnt3, posT, eps, gridc, wd)
    return (loss.reshape(()), ch.reshape(()), kl.reshape(()),
            mu.reshape(B, 64), recon)


# speedup vs baseline: 13.9524x; 12.7039x over previous
import functools

import jax
import jax.numpy as jnp
from jax import lax
from jax.experimental import pallas as pl
from jax.experimental.pallas import tpu as pltpu
from jax.experimental.pallas import tpu_sc as plsc

B, P = 8, 2048
N1, N2, N3 = 1024, 256, 64
KN = 64
G = 2500

INTERPRET = False


def _fps_level(px, py, pz, n_sample):
    Bc, N = px.shape
    lane = lax.broadcasted_iota(jnp.int32, (Bc, N), 1)
    qlane = lax.broadcasted_iota(jnp.int32, (Bc, n_sample), 1)

    def step(i, carry):
        mind, lpx, lpy, lpz, qx, qy, qz = carry
        dx = px - lpx
        dy = py - lpy
        dz = pz - lpz
        d = (dx * dx + dy * dy) + dz * dz
        mind = jnp.minimum(mind, d)
        m = jnp.max(mind, axis=1, keepdims=True)
        cand = jnp.where(mind == m, lane, N)
        j = jnp.min(cand, axis=1, keepdims=True)
        sel = lane == j
        lpx = jnp.sum(jnp.where(sel, px, 0.0), axis=1, keepdims=True)
        lpy = jnp.sum(jnp.where(sel, py, 0.0), axis=1, keepdims=True)
        lpz = jnp.sum(jnp.where(sel, pz, 0.0), axis=1, keepdims=True)
        qx = jnp.where(qlane == i, lpx, qx)
        qy = jnp.where(qlane == i, lpy, qy)
        qz = jnp.where(qlane == i, lpz, qz)
        return (mind, lpx, lpy, lpz, qx, qy, qz)

    zer = jnp.zeros((Bc, n_sample), jnp.float32)
    init = (
        jnp.full((Bc, N), 1e30, jnp.float32),
        px[:, 0:1],
        py[:, 0:1],
        pz[:, 0:1],
        jnp.where(qlane == 0, px[:, 0:1], zer),
        jnp.where(qlane == 0, py[:, 0:1], zer),
        jnp.where(qlane == 0, pz[:, 0:1], zer),
    )
    out = lax.fori_loop(1, n_sample, step, init)
    return out[4], out[5], out[6]


def _fps_kernel(px_ref, py_ref, pz_ref,
                q1x_ref, q1y_ref, q1z_ref,
                q2x_ref, q2y_ref, q2z_ref,
                q3x_ref, q3y_ref, q3z_ref):
    px, py, pz = px_ref[...], py_ref[...], pz_ref[...]
    q1x, q1y, q1z = _fps_level(px, py, pz, N1)
    q1x_ref[...], q1y_ref[...], q1z_ref[...] = q1x, q1y, q1z
    q2x, q2y, q2z = _fps_level(q1x, q1y, q1z, N2)
    q2x_ref[...], q2y_ref[...], q2z_ref[...] = q2x, q2y, q2z
    q3x, q3y, q3z = _fps_level(q2x, q2y, q2z, N3)
    q3x_ref[...], q3y_ref[...], q3z_ref[...] = q3x, q3y, q3z


def _run_fps(px, py, pz):
    sh = lambda n: jax.ShapeDtypeStruct((B, n), jnp.float32)
    return pl.pallas_call(
        _fps_kernel,
        out_shape=(sh(N1), sh(N1), sh(N1), sh(N2), sh(N2), sh(N2),
                   sh(N3), sh(N3), sh(N3)),
        interpret=INTERPRET,
    )(px, py, pz)


def _mlp_body(rel_ref, xg_ref, cnt_ref, w1p_ref, w1x_ref, w2_ref, b1_ref,
              b2_ref, out_ref, *, qb, C):
    rel = rel_ref[0]
    h = jnp.dot(rel, w1p_ref[...], preferred_element_type=jnp.float32)
    if xg_ref is not None:
        h = h + jnp.dot(xg_ref[0], w1x_ref[...],
                        preferred_element_type=jnp.float32)
    h = jax.nn.relu(h + b1_ref[...])
    h = jnp.dot(h, w2_ref[...], preferred_element_type=jnp.float32)
    h = h + b2_ref[...]
    h3 = h.reshape(qb, KN, C)
    kidx = lax.broadcasted_iota(jnp.int32, (qb, KN, C), 1)
    h3 = jnp.where(kidx < cnt_ref[0], h3, -1e9)
    grp = jnp.max(h3, axis=1)
    out_ref[0] = jnp.where(grp <= -1e8, 0.0, grp)


def _run_mlp(rel, xg, cnt, w1p, w1x, w2t, b1, b2, Q, C, qb):
    nq = Q // qb
    pairs = qb * KN
    Cin = 0 if xg is None else xg.shape[-1]

    specs = [pl.BlockSpec((1, pairs, 4), lambda b, q: (b, q, 0))]
    args = [rel]
    if xg is not None:
        specs.append(pl.BlockSpec((1, pairs, Cin), lambda b, q: (b, q, 0)))
        args.append(xg)
    specs.append(pl.BlockSpec((1, qb, 1, 1), lambda b, q: (b, q, 0, 0)))
    args.append(cnt)
    wfull = lambda a: pl.BlockSpec(a.shape, lambda b, q: (0,) * a.ndim)
    for a in (w1p,) + (() if xg is None else (w1x,)) + (w2t, b1, b2):
        specs.append(wfull(a))
        args.append(a)

    body = functools.partial(_mlp_body, qb=qb, C=C)
    if xg is None:
        body2 = lambda rel_ref, cnt_ref, w1p_ref, w2_ref, b1_ref, b2_ref, out_ref: body(
            rel_ref, None, cnt_ref, w1p_ref, None, w2_ref, b1_ref, b2_ref, out_ref)
    else:
        body2 = body

    return pl.pallas_call(
        body2,
        grid=(B, nq),
        in_specs=specs,
        out_specs=pl.BlockSpec((1, qb, C), lambda b, q: (b, q, 0)),
        out_shape=jax.ShapeDtypeStruct((B, Q, C), jnp.float32),
        interpret=INTERPRET,
    )(*args)


def _final_body(xg_ref, rel_ref, cnt_ref, posT_ref, eps_ref, grid_ref,
                w3p_ref, w3x_ref, w3t_ref, b31_ref, b32_ref,
                muw_ref, mub_ref, lvw_ref, lvb_ref,
                f1wz_ref, f1wg_ref, f1b1_ref, f1w2_ref, f1b2_ref, f1w3_ref,
                f1b3_ref,
                f2wz_ref, f2wp_ref, f2b1_ref, f2w2_ref, f2b2_ref, f2w3_ref,
                f2b3_ref,
                mu_ref, recon_ref, loss_ref, ch_ref, kl_ref, acc_ref):
    b = pl.program_id(0)
    f32 = jnp.float32

    h = jnp.dot(rel_ref[0], w3p_ref[...], preferred_element_type=f32)
    h = h + jnp.dot(xg_ref[0], w3x_ref[...], preferred_element_type=f32)
    h = jax.nn.relu(h + b31_ref[...])
    h = jnp.dot(h, w3t_ref[...], preferred_element_type=f32) + b32_ref[...]
    h3 = h.reshape(N3, KN, 256)
    kidx = lax.broadcasted_iota(jnp.int32, (N3, KN, 256), 1)
    h3 = jnp.where(kidx < cnt_ref[0], h3, -1e9)
    x3 = jnp.max(h3, axis=1)
    x3 = jnp.where(x3 <= -1e8, 0.0, x3)
    pooled = jnp.max(x3, axis=0, keepdims=True)

    mu = jnp.dot(pooled, muw_ref[...], preferred_element_type=f32) + mub_ref[...]
    lv = jnp.dot(pooled, lvw_ref[...], preferred_element_type=f32) + lvb_ref[...]
    z = mu + jnp.exp(0.5 * lv) * eps_ref[0]

    gr = grid_ref[...]
    zt = jnp.dot(z, f1wz_ref[...], preferred_element_type=f32)
    h1 = jax.nn.relu(
        jnp.dot(gr, f1wg_ref[...], preferred_element_type=f32) + zt
        + f1b1_ref[...])
    h1 = jax.nn.relu(
        jnp.dot(h1, f1w2_ref[...], preferred_element_type=f32) + f1b2_ref[...])
    x1g = jnp.dot(h1, f1w3_ref[...], preferred_element_type=f32) + f1b3_ref[...]

    zt2 = jnp.dot(z, f2wz_ref[...], preferred_element_type=f32)
    h2 = jax.nn.relu(
        jnp.dot(x1g, f2wp_ref[...], preferred_element_type=f32) + zt2
        + f2b1_ref[...])
    h2 = jax.nn.relu(
        jnp.dot(h2, f2w2_ref[...], preferred_element_type=f32) + f2b2_ref[...])
    recon = jnp.dot(h2, f2w3_ref[...], preferred_element_type=f32) + f2b3_ref[...]

    mu_ref[0] = mu
    recon_ref[0] = recon

    posT = posT_ref[0]
    tsq = jnp.sum(posT * posT, axis=0, keepdims=True)
    psq = jnp.sum(recon * recon, axis=1, keepdims=True)
    cross = jnp.dot(recon, posT, preferred_element_type=f32)
    d2 = psq + tsq - 2.0 * cross
    mA = jnp.min(d2, axis=1)
    mB = jnp.min(d2, axis=0)
    dA = jnp.sqrt(jnp.maximum(mA, 0.0) + 1e-12)
    dB = jnp.sqrt(jnp.maximum(mB, 0.0) + 1e-12)
    ch_part = jnp.sum(dA) / G + jnp.sum(dB) / P

    kl_part = jnp.sum(1.0 + lv - mu * mu - jnp.exp(lv))

    @pl.when(b == 0)
    def _():
        acc_ref[0] = ch_part
        acc_ref[1] = kl_part

    @pl.when(b > 0)
    def _():
        acc_ref[0] += ch_part
        acc_ref[1] += kl_part

    @pl.when(b == B - 1)
    def _():
        ch = acc_ref[0] / B
        kl = -0.5 * acc_ref[1] / B
        ch_ref[0, 0] = ch
        kl_ref[0, 0] = kl
        loss_ref[0, 0] = ch + 0.001 * kl


def _run_final(xg3, rel3, cnt3, posT, eps, gridc, wd):
    wfull = lambda a: pl.BlockSpec(a.shape, lambda b: (0,) * a.ndim)
    specs = [
        pl.BlockSpec((1, N3 * KN, 128), lambda b: (b, 0, 0)),
        pl.BlockSpec((1, N3 * KN, 4), lambda b: (b, 0, 0)),
        pl.BlockSpec((1, N3, 1, 1), lambda b: (b, 0, 0, 0)),
        pl.BlockSpec((1, 3, P), lambda b: (b, 0, 0)),
        pl.BlockSpec((1, 1, 64), lambda b: (b, 0, 0)),
        wfull(gridc),
    ]
    args = [xg3, rel3, cnt3, posT, eps, gridc]
    for a in wd:
        specs.append(wfull(a))
        args.append(a)
    sm = pltpu.SMEM
    out_shape = (
        jax.ShapeDtypeStruct((B, 1, 64), jnp.float32),
        jax.ShapeDtypeStruct((B, G, 3), jnp.float32),
        jax.ShapeDtypeStruct((1, 1), jnp.float32),
        jax.ShapeDtypeStruct((1, 1), jnp.float32),
        jax.ShapeDtypeStruct((1, 1), jnp.float32),
    )
    out_specs = (
        pl.BlockSpec((1, 1, 64), lambda b: (b, 0, 0)),
        pl.BlockSpec((1, G, 3), lambda b: (b, 0, 0)),
        pl.BlockSpec(memory_space=sm),
        pl.BlockSpec(memory_space=sm),
        pl.BlockSpec(memory_space=sm),
    )
    return pl.pallas_call(
        _final_body,
        grid=(B,),
        in_specs=specs,
        out_specs=out_specs,
        out_shape=out_shape,
        scratch_shapes=[pltpu.SMEM((2,), jnp.float32)],
        interpret=INTERPRET,
    )(*args)


def _pack_body(qx_ref, qy_ref, qz_ref, px_ref, py_ref, pz_ref, w_ref, o_ref,
               *, thr):
    qx, qy, qz = qx_ref[0], qy_ref[0], qz_ref[0]
    px, py, pz = px_ref[0], py_ref[0], pz_ref[0]
    dx = qx - px
    dy = qy - py
    dz = qz - pz
    d2 = (dx * dx + dy * dy) + dz * dz
    mask = (d2 <= thr).astype(jnp.float32)
    o_ref[0] = jnp.dot(mask, w_ref[...], preferred_element_type=jnp.float32)


def _run_pack(qc, pc, r, Q, N, qb):
    NW = N // 16
    n = jnp.arange(N, dtype=jnp.int32)
    W = jnp.where(n[:, None] // 16 == jnp.arange(NW, dtype=jnp.int32)[None, :],
                  jnp.exp2((n % 16).astype(jnp.float32))[:, None], 0.0)
    col = lambda a: pl.BlockSpec((1, qb, 1), lambda b, q: (b, q, 0))
    row = lambda a: pl.BlockSpec((1, 1, N), lambda b, q: (b, 0, 0))
    specs = [col(a) for a in qc] + [row(a) for a in pc]
    specs.append(pl.BlockSpec((N, NW), lambda b, q: (0, 0)))
    thr = float(r) * float(r)
    return pl.pallas_call(
        functools.partial(_pack_body, thr=thr),
        grid=(B, Q // qb),
        in_specs=specs,
        out_specs=pl.BlockSpec((1, qb, NW), lambda b, q: (b, q, 0)),
        out_shape=jax.ShapeDtypeStruct((B, Q, NW), jnp.float32),
        interpret=INTERPRET,
    )(*qc, *pc, W)


def _make_compact(NQ, NPT, QC, stages=7):
    NW = NPT // 16
    QW = NQ // 4
    NCH = QW // QC
    f32, i32 = jnp.float32, jnp.int32
    mesh = plsc.VectorSubcoreMesh(core_axis_name="c", subcore_axis_name="s")

    @functools.partial(
        pl.kernel,
        mesh=mesh,
        compiler_params=pltpu.CompilerParams(use_tc_tiling_on_sc=False, needs_layout_passes=False),
        out_type=(
            jax.ShapeDtypeStruct((B, NQ * KN, 4), f32),
            jax.ShapeDtypeStruct((B, NQ, KN), i32),
            jax.ShapeDtypeStruct((B, NQ), i32),
        ),
        scratch_types=[
            pltpu.VMEM((NPT,), f32),
            pltpu.VMEM((NPT,), f32),
            pltpu.VMEM((NPT,), f32),
            pltpu.VMEM((QW,), f32),
            pltpu.VMEM((QW,), f32),
            pltpu.VMEM((QW,), f32),
            pltpu.VMEM((QC, NW), f32),
            pltpu.VMEM((NW + 16,), i32),
            pltpu.VMEM((QC * 80,), i32),
            pltpu.VMEM((QC, KN), i32),
            pltpu.VMEM((QC * KN, 4), f32),
            pltpu.VMEM((QC,), i32),
        ],
    )
    def kern(pk, ptx, pty, ptz, qxh, qyh, qzh,
             rel_o, nbr_o, cnt_o,
             vx, vy, vz, vqx, vqy, vqz, vpk, wbuf, nbst, nbg, relv, cntv):
        cc = lax.axis_index("c")
        ss = lax.axis_index("s")
        wid = ss * 2 + cc
        b = wid // 4
        qw0 = (wid % 4) * QW
        pltpu.sync_copy(ptx.at[b], vx)
        pltpu.sync_copy(pty.at[b], vy)
        pltpu.sync_copy(ptz.at[b], vz)
        pltpu.sync_copy(qxh.at[b, pl.ds(qw0, QW)], vqx)
        pltpu.sync_copy(qyh.at[b, pl.ds(qw0, QW)], vqy)
        pltpu.sync_copy(qzh.at[b, pl.ds(qw0, QW)], vqz)

        iota16 = lax.iota(i32, 16)
        z16 = jnp.zeros((16,), i32)
        zf16 = jnp.zeros((16,), f32)
        bN = b * NPT

        def chunk_body(ch, _):
            q0 = qw0 + ch * QC
            if stages >= -1:
                pltpu.sync_copy(pk.at[b, pl.ds(q0, QC), :], vpk)

            def q_body(qi, _2):
                qis0 = jnp.full((16,), qi, i32)
                if stages < 0:
                    return 0
                for kc in range(4):
                    nbst[pl.ds(qi * 80 + kc * 16, 16)] = z16
                if stages < 1:
                    return 0
                if stages >= 10:
                    wv = vpk[qi, pl.ds(0, 16)]
                    m = wv != 0.0
                    if stages == 11:
                        cs = plsc.cumsum(m.astype(i32))
                    elif stages == 12:
                        cs = jnp.full((16,), jnp.sum(m.astype(i32)), i32)
                    elif stages == 13:
                        cs = plsc.sort_key_val(
                            jnp.where(m, iota16, iota16 + 16), iota16)
                    elif stages == 14:
                        cs = plsc.all_reduce_population_count(m)
                    elif stages == 15:
                        cs = jnp.cumsum(m.astype(i32))
                    else:
                        cs = iota16
                    if not isinstance(cs, tuple):
                        cs = (cs,)
                    wbuf[pl.ds(0, 16)] = cs[-1].astype(i32)
                    return 0
                nzc = jnp.int32(0)
                for wc in range(NW // 16):
                    wv = vpk[qi, pl.ds(wc * 16, 16)]
                    m = wv != 0.0
                    cs = plsc.cumsum(m.astype(i32))
                    plsc.store_scatter(wbuf, [nzc + cs - 1],
                                       iota16 + wc * 16, mask=m)
                    nzc = nzc + jnp.sum(m.astype(i32))

                qis = jnp.full((16,), qi, i32)
                if stages < 2:
                    return 0

                def w_body(j, cursor):
                    js = jnp.full((16,), j, i32)
                    wids = plsc.load_gather(wbuf, [js])
                    wval = plsc.load_gather(vpk, [qis, wids])
                    wi = wval.astype(i32)
                    m = ((wi >> iota16) & 1) == 1
                    csum = plsc.cumsum(m.astype(i32))
                    m = m & ((cursor + csum) <= KN)
                    plsc.store_scatter(nbst, [qi * 80 + cursor + csum - 1],
                                       wids * 16 + iota16, mask=m)
                    return cursor + jnp.sum(m.astype(i32))

                cursor = lax.fori_loop(0, nzc, w_body, jnp.int32(0))
                plsc.store_scatter(cntv, [qis], jnp.full((16,), cursor, i32),
                                   mask=iota16 == 0)
                if stages < 3:
                    return 0

                qls = jnp.full((16,), ch * QC + qi, i32)
                qxs = plsc.load_gather(vqx, [qls])
                qys = plsc.load_gather(vqy, [qls])
                qzs = plsc.load_gather(vqz, [qls])
                for kc in range(4):
                    idx = nbst[pl.ds(qi * 80 + kc * 16, 16)]
                    gx = plsc.load_gather(vx, [idx]) - qxs
                    gy = plsc.load_gather(vy, [idx]) - qys
                    gz = plsc.load_gather(vz, [idx]) - qzs
                    rows = qi * KN + kc * 16 + iota16
                    plsc.store_scatter(relv, [rows, z16], gx)
                    plsc.store_scatter(relv, [rows, z16 + 1], gy)
                    plsc.store_scatter(relv, [rows, z16 + 2], gz)
                    plsc.store_scatter(relv, [rows, z16 + 3], zf16)
                    plsc.store_scatter(nbg, [qis, kc * 16 + iota16],
                                       idx + bN)
                return 0

            if stages >= -2:
                lax.fori_loop(0, QC, q_body, 0)
            pltpu.sync_copy(relv, rel_o.at[b, pl.ds(q0 * KN, QC * KN), :])
            pltpu.sync_copy(nbg, nbr_o.at[b, pl.ds(q0, QC), :])
            pltpu.sync_copy(cntv, cnt_o.at[b, pl.ds(q0, QC)])
            return 0

        lax.fori_loop(0, NCH, chunk_body, 0)

    return kern


def _make_rowgather(NQ, C, QCH):
    f32, i32 = jnp.float32, jnp.int32
    QW = NQ // 4
    NCH = QW // QCH
    mesh = plsc.VectorSubcoreMesh(core_axis_name="c", subcore_axis_name="s")

    @functools.partial(
        pl.kernel,
        mesh=mesh,
        compiler_params=pltpu.CompilerParams(use_tc_tiling_on_sc=False,
                                             needs_layout_passes=False),
        out_type=jax.ShapeDtypeStruct((B * NQ * KN, C), f32),
        scratch_types=[
            pltpu.VMEM((QW, KN), i32),
            pltpu.VMEM((QCH * KN, C), f32),
            pltpu.SemaphoreType.DMA,
        ],
    )
    def kern(xflat, nbr, out, vidx, rows, sem):
        cc = lax.axis_index("c")
        ss = lax.axis_index("s")
        wid = ss * 2 + cc
        b = wid // 4
        q0 = (wid % 4) * QW
        pltpu.sync_copy(nbr.at[b, pl.ds(q0, QW), :], vidx)

        def chunk_body(ch, _):
            cps = []
            for j in range(QCH):
                cp = pltpu.make_async_copy(
                    xflat.at[vidx.at[ch * QCH + j]],
                    rows.at[pl.ds(j * KN, KN), :],
                    sem,
                )
                cp.start()
                cps.append(cp)
            for cp in cps:
                cp.wait()
            row0 = (b * NQ + q0 + ch * QCH) * KN
            pltpu.sync_copy(rows, out.at[pl.ds(row0, QCH * KN), :])
            return 0

        lax.fori_loop(0, NCH, chunk_body, 0)

    return kern


def _select_gather(qx, qy, qz, px, py, pz, r, feats=None):
    N = px.shape[-1]
    Q = qx.shape[-1]

    def one(qx1, qy1, qz1, px1, py1, pz1, f1):
        dx = qx1[:, None] - px1[None, :]
        dy = qy1[:, None] - py1[None, :]
        dz = qz1[:, None] - pz1[None, :]
        d2 = (dx * dx + dy * dy) + dz * dz
        mask = d2 <= r * r
        iota = jnp.arange(N, dtype=jnp.int32)
        selv = jnp.where(mask, -iota, -N - 1)
        vals, _ = lax.top_k(selv, KN)
        got = vals > -N - 1
        nbr = jnp.where(got, -vals, 0)
        cnt = jnp.sum(mask.astype(jnp.int32), axis=1)
        cnt = jnp.minimum(cnt, KN)
        relx = px1[nbr] - qx1[:, None]
        rely = py1[nbr] - qy1[:, None]
        relz = pz1[nbr] - qz1[:, None]
        rel = jnp.stack(
            [relx, rely, relz, jnp.zeros_like(relx)], axis=-1
        ).reshape(Q * KN, 4)
        fg = None if f1 is None else f1[nbr].reshape(Q * KN, -1)
        return rel, cnt.reshape(Q, 1, 1), fg

    return jax.vmap(one)(qx, qy, qz, px, py, pz, feats)


def _tw(w):
    return jnp.asarray(w.T, jnp.float32)


def _padt(w, k=4):
    wt = w.T
    return jnp.pad(wt, ((0, k - wt.shape[0]), (0, 0)))


def kernel(pos, params, batch):
    pr = pos.reshape(B, P, 3)
    px = pr[:, :, 0]
    py = pr[:, :, 1]
    pz = pr[:, :, 2]

    (q1x, q1y, q1z, q2x, q2y, q2z, q3x, q3y, q3z) = _run_fps(px, py, pz)

    p = params
    col = lambda ax, n: [a.reshape(B, n, 1) for a in ax]
    rowv = lambda ax, n: [a.reshape(B, 1, n) for a in ax]
    q1 = (q1x, q1y, q1z)
    q2 = (q2x, q2y, q2z)
    q3 = (q3x, q3y, q3z)
    pp = (px, py, pz)

    pk1 = _run_pack(col(q1, N1), rowv(pp, P), 0.2, N1, P, 256)
    pk2 = _run_pack(col(q2, N2), rowv(q1, N1), 0.4, N2, N1, 256)
    pk3 = _run_pack(col(q3, N3), rowv(q2, N2), 0.8, N3, N2, 64)

    rel1, nbr1, cnt1 = _make_compact(N1, P, 64)(pk1, *pp, *q1)
    rel2, nbr2, cnt2 = _make_compact(N2, N1, 64)(pk2, *q1, *q2)
    rel3, nbr3, cnt3 = _make_compact(N3, N2, 16)(pk3, *q2, *q3)

    x1 = _run_mlp(rel1, None, cnt1.reshape(B, N1, 1, 1), _padt(p["sa1_w1"]),
                  None, _tw(p["sa1_w2"]), p["sa1_b1"][None], p["sa1_b2"][None],
                  N1, 64, 128)

    xg2 = _make_rowgather(N2, 64, 16)(x1.reshape(B * N1, 64), nbr2)
    xg2 = xg2.reshape(B, N2 * KN, 64)
    x2 = _run_mlp(rel2, xg2, cnt2.reshape(B, N2, 1, 1),
                  _padt(p["sa2_w1"][:, 64:]),
                  _tw(p["sa2_w1"][:, :64]), _tw(p["sa2_w2"]),
                  p["sa2_b1"][None], p["sa2_b2"][None], N2, 128, 128)

    xg3 = _make_rowgather(N3, 128, 8)(x2.reshape(B * N2, 128), nbr3)
    xg3 = xg3.reshape(B, N3 * KN, 128)
    cnt3 = cnt3.reshape(B, N3, 1, 1)

    eps = jax.random.normal(jax.random.key(42), (B, 64), dtype=jnp.float32)
    eps = eps.reshape(B, 1, 64)
    xs = jnp.linspace(-0.3, 0.3, 50)
    gx, gy = jnp.meshgrid(xs, xs, indexing="ij")
    gridc = jnp.stack([gx.ravel(), gy.ravel()], axis=-1).astype(jnp.float32)

    posT = jnp.stack([px, py, pz], axis=1)

    wd = [
        _padt(p["sa3_w1"][:, 128:]), _tw(p["sa3_w1"][:, :128]),
        _tw(p["sa3_w2"]), p["sa3_b1"][None], p["sa3_b2"][None],
        _tw(p["mu_w"]), p["mu_b"][None], _tw(p["lv_w"]), p["lv_b"][None],
        _tw(p["f1_w1"][:, :64]), _tw(p["f1_w1"][:, 64:]), p["f1_b1"][None],
        _tw(p["f1_w2"]), p["f1_b2"][None], _tw(p["f1_w3"]), p["f1_b3"][None],
        _tw(p["f2_w1"][:, :64]), _tw(p["f2_w1"][:, 64:]), p["f2_b1"][None],
        _tw(p["f2_w2"]), p["f2_b2"][None], _tw(p["f2_w3"]), p["f2_b3"][None],
    ]
    mu, recon, loss, ch, kl = _run_final(xg3, rel3, cnt3, posT, eps, gridc, wd)
    return (loss.reshape(()), ch.reshape(()), kl.reshape(()),
            mu.reshape(B, 64), recon)


# speedup vs baseline: 14.0167x; 1.0046x over previous
import functools

import jax
import jax.numpy as jnp
from jax import lax
from jax.experimental import pallas as pl
from jax.experimental.pallas import tpu as pltpu
from jax.experimental.pallas import tpu_sc as plsc

B, P = 8, 2048
N1, N2, N3 = 1024, 256, 64
KN = 64
G = 2500

INTERPRET = False


def _fps_level(px, py, pz, n_sample):
    Bc, N = px.shape
    lane = lax.broadcasted_iota(jnp.int32, (Bc, N), 1)
    qlane = lax.broadcasted_iota(jnp.int32, (Bc, n_sample), 1)

    def step(i, carry):
        mind, lpx, lpy, lpz, qx, qy, qz = carry
        dx = px - lpx
        dy = py - lpy
        dz = pz - lpz
        d = (dx * dx + dy * dy) + dz * dz
        mind = jnp.minimum(mind, d)
        m = jnp.max(mind, axis=1, keepdims=True)
        cand = jnp.where(mind == m, lane, N)
        j = jnp.min(cand, axis=1, keepdims=True)
        sel = lane == j
        lpx = jnp.sum(jnp.where(sel, px, 0.0), axis=1, keepdims=True)
        lpy = jnp.sum(jnp.where(sel, py, 0.0), axis=1, keepdims=True)
        lpz = jnp.sum(jnp.where(sel, pz, 0.0), axis=1, keepdims=True)
        qx = jnp.where(qlane == i, lpx, qx)
        qy = jnp.where(qlane == i, lpy, qy)
        qz = jnp.where(qlane == i, lpz, qz)
        return (mind, lpx, lpy, lpz, qx, qy, qz)

    zer = jnp.zeros((Bc, n_sample), jnp.float32)
    init = (
        jnp.full((Bc, N), 1e30, jnp.float32),
        px[:, 0:1],
        py[:, 0:1],
        pz[:, 0:1],
        jnp.where(qlane == 0, px[:, 0:1], zer),
        jnp.where(qlane == 0, py[:, 0:1], zer),
        jnp.where(qlane == 0, pz[:, 0:1], zer),
    )
    out = lax.fori_loop(1, n_sample, step, init)
    return out[4], out[5], out[6]


def _fps_kernel(px_ref, py_ref, pz_ref,
                q1x_ref, q1y_ref, q1z_ref,
                q2x_ref, q2y_ref, q2z_ref,
                q3x_ref, q3y_ref, q3z_ref):
    px, py, pz = px_ref[...], py_ref[...], pz_ref[...]
    q1x, q1y, q1z = _fps_level(px, py, pz, N1)
    q1x_ref[...], q1y_ref[...], q1z_ref[...] = q1x, q1y, q1z
    q2x, q2y, q2z = _fps_level(q1x, q1y, q1z, N2)
    q2x_ref[...], q2y_ref[...], q2z_ref[...] = q2x, q2y, q2z
    q3x, q3y, q3z = _fps_level(q2x, q2y, q2z, N3)
    q3x_ref[...], q3y_ref[...], q3z_ref[...] = q3x, q3y, q3z


def _run_fps(px, py, pz):
    sh = lambda n: jax.ShapeDtypeStruct((B, n), jnp.float32)
    return pl.pallas_call(
        _fps_kernel,
        out_shape=(sh(N1), sh(N1), sh(N1), sh(N2), sh(N2), sh(N2),
                   sh(N3), sh(N3), sh(N3)),
        interpret=INTERPRET,
    )(px, py, pz)


def _mlp_body(rel_ref, xg_ref, cnt_ref, w1p_ref, w1x_ref, w2_ref, b1_ref,
              b2_ref, out_ref, *, qb, C):
    rel = rel_ref[0]
    h = jnp.dot(rel, w1p_ref[...], preferred_element_type=jnp.float32)
    if xg_ref is not None:
        h = h + jnp.dot(xg_ref[0], w1x_ref[...],
                        preferred_element_type=jnp.float32)
    h = jax.nn.relu(h + b1_ref[...])
    h = jnp.dot(h, w2_ref[...], preferred_element_type=jnp.float32)
    h = h + b2_ref[...]
    h3 = h.reshape(qb, KN, C)
    kidx = lax.broadcasted_iota(jnp.int32, (qb, KN, C), 1)
    h3 = jnp.where(kidx < cnt_ref[0], h3, -1e9)
    grp = jnp.max(h3, axis=1)
    out_ref[0] = jnp.where(grp <= -1e8, 0.0, grp)


def _run_mlp(rel, xg, cnt, w1p, w1x, w2t, b1, b2, Q, C, qb):
    nq = Q // qb
    pairs = qb * KN
    Cin = 0 if xg is None else xg.shape[-1]

    specs = [pl.BlockSpec((1, pairs, 4), lambda b, q: (b, q, 0))]
    args = [rel]
    if xg is not None:
        specs.append(pl.BlockSpec((1, pairs, Cin), lambda b, q: (b, q, 0)))
        args.append(xg)
    specs.append(pl.BlockSpec((1, qb, 1, 1), lambda b, q: (b, q, 0, 0)))
    args.append(cnt)
    wfull = lambda a: pl.BlockSpec(a.shape, lambda b, q: (0,) * a.ndim)
    for a in (w1p,) + (() if xg is None else (w1x,)) + (w2t, b1, b2):
        specs.append(wfull(a))
        args.append(a)

    body = functools.partial(_mlp_body, qb=qb, C=C)
    if xg is None:
        body2 = lambda rel_ref, cnt_ref, w1p_ref, w2_ref, b1_ref, b2_ref, out_ref: body(
            rel_ref, None, cnt_ref, w1p_ref, None, w2_ref, b1_ref, b2_ref, out_ref)
    else:
        body2 = body

    return pl.pallas_call(
        body2,
        grid=(B, nq),
        in_specs=specs,
        out_specs=pl.BlockSpec((1, qb, C), lambda b, q: (b, q, 0)),
        out_shape=jax.ShapeDtypeStruct((B, Q, C), jnp.float32),
        interpret=INTERPRET,
    )(*args)


def _final_body(xg_ref, rel_ref, cnt_ref, posT_ref, eps_ref, grid_ref,
                w3p_ref, w3x_ref, w3t_ref, b31_ref, b32_ref,
                muw_ref, mub_ref, lvw_ref, lvb_ref,
                f1wz_ref, f1wg_ref, f1b1_ref, f1w2_ref, f1b2_ref, f1w3_ref,
                f1b3_ref,
                f2wz_ref, f2wp_ref, f2b1_ref, f2w2_ref, f2b2_ref, f2w3_ref,
                f2b3_ref,
                mu_ref, recon_ref, loss_ref, ch_ref, kl_ref, acc_ref):
    b = pl.program_id(0)
    f32 = jnp.float32

    h = jnp.dot(rel_ref[0], w3p_ref[...], preferred_element_type=f32)
    h = h + jnp.dot(xg_ref[0], w3x_ref[...], preferred_element_type=f32)
    h = jax.nn.relu(h + b31_ref[...])
    h = jnp.dot(h, w3t_ref[...], preferred_element_type=f32) + b32_ref[...]
    h3 = h.reshape(N3, KN, 256)
    kidx = lax.broadcasted_iota(jnp.int32, (N3, KN, 256), 1)
    h3 = jnp.where(kidx < cnt_ref[0], h3, -1e9)
    x3 = jnp.max(h3, axis=1)
    x3 = jnp.where(x3 <= -1e8, 0.0, x3)
    pooled = jnp.max(x3, axis=0, keepdims=True)

    mu = jnp.dot(pooled, muw_ref[...], preferred_element_type=f32) + mub_ref[...]
    lv = jnp.dot(pooled, lvw_ref[...], preferred_element_type=f32) + lvb_ref[...]
    z = mu + jnp.exp(0.5 * lv) * eps_ref[0]

    gr = grid_ref[...]
    zt = jnp.dot(z, f1wz_ref[...], preferred_element_type=f32)
    h1 = jax.nn.relu(
        jnp.dot(gr, f1wg_ref[...], preferred_element_type=f32) + zt
        + f1b1_ref[...])
    h1 = jax.nn.relu(
        jnp.dot(h1, f1w2_ref[...], preferred_element_type=f32) + f1b2_ref[...])
    x1g = jnp.dot(h1, f1w3_ref[...], preferred_element_type=f32) + f1b3_ref[...]

    zt2 = jnp.dot(z, f2wz_ref[...], preferred_element_type=f32)
    h2 = jax.nn.relu(
        jnp.dot(x1g, f2wp_ref[...], preferred_element_type=f32) + zt2
        + f2b1_ref[...])
    h2 = jax.nn.relu(
        jnp.dot(h2, f2w2_ref[...], preferred_element_type=f32) + f2b2_ref[...])
    recon = jnp.dot(h2, f2w3_ref[...], preferred_element_type=f32) + f2b3_ref[...]

    mu_ref[0] = mu
    recon_ref[0] = recon

    posT = posT_ref[0]
    tsq = jnp.sum(posT * posT, axis=0, keepdims=True)
    psq = jnp.sum(recon * recon, axis=1, keepdims=True)
    cross = jnp.dot(recon, posT, preferred_element_type=f32)
    d2 = psq + tsq - 2.0 * cross
    mA = jnp.min(d2, axis=1)
    mB = jnp.min(d2, axis=0)
    dA = jnp.sqrt(jnp.maximum(mA, 0.0) + 1e-12)
    dB = jnp.sqrt(jnp.maximum(mB, 0.0) + 1e-12)
    ch_part = jnp.sum(dA) / G + jnp.sum(dB) / P

    kl_part = jnp.sum(1.0 + lv - mu * mu - jnp.exp(lv))

    @pl.when(b == 0)
    def _():
        acc_ref[0] = ch_part
        acc_ref[1] = kl_part

    @pl.when(b > 0)
    def _():
        acc_ref[0] += ch_part
        acc_ref[1] += kl_part

    @pl.when(b == B - 1)
    def _():
        ch = acc_ref[0] / B
        kl = -0.5 * acc_ref[1] / B
        ch_ref[0, 0] = ch
        kl_ref[0, 0] = kl
        loss_ref[0, 0] = ch + 0.001 * kl


def _run_final(xg3, rel3, cnt3, posT, eps, gridc, wd):
    wfull = lambda a: pl.BlockSpec(a.shape, lambda b: (0,) * a.ndim)
    specs = [
        pl.BlockSpec((1, N3 * KN, 128), lambda b: (b, 0, 0)),
        pl.BlockSpec((1, N3 * KN, 4), lambda b: (b, 0, 0)),
        pl.BlockSpec((1, N3, 1, 1), lambda b: (b, 0, 0, 0)),
        pl.BlockSpec((1, 3, P), lambda b: (b, 0, 0)),
        pl.BlockSpec((1, 1, 64), lambda b: (b, 0, 0)),
        wfull(gridc),
    ]
    args = [xg3, rel3, cnt3, posT, eps, gridc]
    for a in wd:
        specs.append(wfull(a))
        args.append(a)
    sm = pltpu.SMEM
    out_shape = (
        jax.ShapeDtypeStruct((B, 1, 64), jnp.float32),
        jax.ShapeDtypeStruct((B, G, 3), jnp.float32),
        jax.ShapeDtypeStruct((1, 1), jnp.float32),
        jax.ShapeDtypeStruct((1, 1), jnp.float32),
        jax.ShapeDtypeStruct((1, 1), jnp.float32),
    )
    out_specs = (
        pl.BlockSpec((1, 1, 64), lambda b: (b, 0, 0)),
        pl.BlockSpec((1, G, 3), lambda b: (b, 0, 0)),
        pl.BlockSpec(memory_space=sm),
        pl.BlockSpec(memory_space=sm),
        pl.BlockSpec(memory_space=sm),
    )
    return pl.pallas_call(
        _final_body,
        grid=(B,),
        in_specs=specs,
        out_specs=out_specs,
        out_shape=out_shape,
        scratch_shapes=[pltpu.SMEM((2,), jnp.float32)],
        interpret=INTERPRET,
    )(*args)


def _pack_body(qx_ref, qy_ref, qz_ref, px_ref, py_ref, pz_ref, w_ref, o_ref,
               *, thr):
    qx, qy, qz = qx_ref[0], qy_ref[0], qz_ref[0]
    px, py, pz = px_ref[0], py_ref[0], pz_ref[0]
    dx = qx - px
    dy = qy - py
    dz = qz - pz
    d2 = (dx * dx + dy * dy) + dz * dz
    mask = (d2 <= thr).astype(jnp.float32)
    o_ref[0] = jnp.dot(mask, w_ref[...], preferred_element_type=jnp.float32)


def _run_pack(qc, pc, r, Q, N, qb):
    NW = N // 16
    n = jnp.arange(N, dtype=jnp.int32)
    W = jnp.where(n[:, None] // 16 == jnp.arange(NW, dtype=jnp.int32)[None, :],
                  jnp.exp2((n % 16).astype(jnp.float32))[:, None], 0.0)
    col = lambda a: pl.BlockSpec((1, qb, 1), lambda b, q: (b, q, 0))
    row = lambda a: pl.BlockSpec((1, 1, N), lambda b, q: (b, 0, 0))
    specs = [col(a) for a in qc] + [row(a) for a in pc]
    specs.append(pl.BlockSpec((N, NW), lambda b, q: (0, 0)))
    thr = float(r) * float(r)
    return pl.pallas_call(
        functools.partial(_pack_body, thr=thr),
        grid=(B, Q // qb),
        in_specs=specs,
        out_specs=pl.BlockSpec((1, qb, NW), lambda b, q: (b, q, 0)),
        out_shape=jax.ShapeDtypeStruct((B, Q, NW), jnp.float32),
        interpret=INTERPRET,
    )(*qc, *pc, W)


def _make_compact(NQ, NPT, QC, unroll=4):
    NW = NPT // 16
    QW = NQ // 4
    NCH = QW // QC
    f32, i32 = jnp.float32, jnp.int32
    mesh = plsc.VectorSubcoreMesh(core_axis_name="c", subcore_axis_name="s")

    @functools.partial(
        pl.kernel,
        mesh=mesh,
        compiler_params=pltpu.CompilerParams(use_tc_tiling_on_sc=False, needs_layout_passes=False),
        out_type=(
            jax.ShapeDtypeStruct((B, NQ * KN, 4), f32),
            jax.ShapeDtypeStruct((B, NQ, KN), i32),
            jax.ShapeDtypeStruct((B, NQ), i32),
        ),
        scratch_types=[
            pltpu.VMEM((NPT,), f32),
            pltpu.VMEM((NPT,), f32),
            pltpu.VMEM((NPT,), f32),
            pltpu.VMEM((QW,), f32),
            pltpu.VMEM((QW,), f32),
            pltpu.VMEM((QW,), f32),
            pltpu.VMEM((QC, NW), f32),
            pltpu.VMEM((QC * (NW + 16),), i32),
            pltpu.VMEM((QC * 80,), i32),
            pltpu.VMEM((QC, KN), i32),
            pltpu.VMEM((QC * KN, 4), f32),
            pltpu.VMEM((QC,), i32),
        ],
    )
    def kern(pk, ptx, pty, ptz, qxh, qyh, qzh,
             rel_o, nbr_o, cnt_o,
             vx, vy, vz, vqx, vqy, vqz, vpk, wbuf, nbst, nbg, relv, cntv):
        cc = lax.axis_index("c")
        ss = lax.axis_index("s")
        wid = ss * 2 + cc
        b = wid // 4
        qw0 = (wid % 4) * QW
        pltpu.sync_copy(ptx.at[b], vx)
        pltpu.sync_copy(pty.at[b], vy)
        pltpu.sync_copy(ptz.at[b], vz)
        pltpu.sync_copy(qxh.at[b, pl.ds(qw0, QW)], vqx)
        pltpu.sync_copy(qyh.at[b, pl.ds(qw0, QW)], vqy)
        pltpu.sync_copy(qzh.at[b, pl.ds(qw0, QW)], vqz)

        iota16 = lax.iota(i32, 16)
        z16 = jnp.zeros((16,), i32)
        zf16 = jnp.zeros((16,), f32)
        bN = b * NPT

        NW16 = NW + 16

        def chunk_body(ch, _):
            q0 = qw0 + ch * QC
            pltpu.sync_copy(pk.at[b, pl.ds(q0, QC), :], vpk)

            @plsc.parallel_loop(0, QC, 1, unroll=unroll)
            def q_body(qi):
                for kc in range(4):
                    nbst[pl.ds(qi * 80 + kc * 16, 16)] = z16
                nzc = jnp.int32(0)
                for wc in range(NW // 16):
                    wv = vpk[qi, pl.ds(wc * 16, 16)]
                    m = wv != 0.0
                    cs = plsc.cumsum(m.astype(i32))
                    plsc.store_scatter(wbuf, [qi * NW16 + nzc + cs - 1],
                                       iota16 + wc * 16, mask=m)
                    nzc = nzc + jnp.sum(m.astype(i32))

                qis = jnp.full((16,), qi, i32)

                def w_body(j, cursor):
                    js = jnp.full((16,), qi * NW16 + j, i32)
                    wids = plsc.load_gather(wbuf, [js])
                    wval = plsc.load_gather(vpk, [qis, wids])
                    wi = wval.astype(i32)
                    m = ((wi >> iota16) & 1) == 1
                    csum = plsc.cumsum(m.astype(i32))
                    m = m & ((cursor + csum) <= KN)
                    plsc.store_scatter(nbst, [qi * 80 + cursor + csum - 1],
                                       wids * 16 + iota16, mask=m)
                    return cursor + jnp.sum(m.astype(i32))

                cursor = lax.fori_loop(0, nzc, w_body, jnp.int32(0))
                plsc.store_scatter(cntv, [qis], jnp.full((16,), cursor, i32),
                                   mask=iota16 == 0)

                qls = jnp.full((16,), ch * QC + qi, i32)
                qxs = plsc.load_gather(vqx, [qls])
                qys = plsc.load_gather(vqy, [qls])
                qzs = plsc.load_gather(vqz, [qls])
                for kc in range(4):
                    idx = nbst[pl.ds(qi * 80 + kc * 16, 16)]
                    gx = plsc.load_gather(vx, [idx]) - qxs
                    gy = plsc.load_gather(vy, [idx]) - qys
                    gz = plsc.load_gather(vz, [idx]) - qzs
                    rows = qi * KN + kc * 16 + iota16
                    plsc.store_scatter(relv, [rows, z16], gx)
                    plsc.store_scatter(relv, [rows, z16 + 1], gy)
                    plsc.store_scatter(relv, [rows, z16 + 2], gz)
                    plsc.store_scatter(relv, [rows, z16 + 3], zf16)
                    plsc.store_scatter(nbg, [qis, kc * 16 + iota16],
                                       idx + bN)

            pltpu.sync_copy(relv, rel_o.at[b, pl.ds(q0 * KN, QC * KN), :])
            pltpu.sync_copy(nbg, nbr_o.at[b, pl.ds(q0, QC), :])
            pltpu.sync_copy(cntv, cnt_o.at[b, pl.ds(q0, QC)])
            return 0

        lax.fori_loop(0, NCH, chunk_body, 0)

    return kern


def _make_rowgather(NQ, C, QCH):
    f32, i32 = jnp.float32, jnp.int32
    QW = NQ // 4
    NCH = QW // QCH
    mesh = plsc.VectorSubcoreMesh(core_axis_name="c", subcore_axis_name="s")

    @functools.partial(
        pl.kernel,
        mesh=mesh,
        compiler_params=pltpu.CompilerParams(use_tc_tiling_on_sc=False,
                                             needs_layout_passes=False),
        out_type=jax.ShapeDtypeStruct((B * NQ * KN, C), f32),
        scratch_types=[
            pltpu.VMEM((QW, KN), i32),
            pltpu.VMEM((QCH * KN, C), f32),
            pltpu.SemaphoreType.DMA,
        ],
    )
    def kern(xflat, nbr, out, vidx, rows, sem):
        cc = lax.axis_index("c")
        ss = lax.axis_index("s")
        wid = ss * 2 + cc
        b = wid // 4
        q0 = (wid % 4) * QW
        pltpu.sync_copy(nbr.at[b, pl.ds(q0, QW), :], vidx)

        def chunk_body(ch, _):
            cps = []
            for j in range(QCH):
                cp = pltpu.make_async_copy(
                    xflat.at[vidx.at[ch * QCH + j]],
                    rows.at[pl.ds(j * KN, KN), :],
                    sem,
                )
                cp.start()
                cps.append(cp)
            for cp in cps:
                cp.wait()
            row0 = (b * NQ + q0 + ch * QCH) * KN
            pltpu.sync_copy(rows, out.at[pl.ds(row0, QCH * KN), :])
            return 0

        lax.fori_loop(0, NCH, chunk_body, 0)

    return kern


def _select_gather(qx, qy, qz, px, py, pz, r, feats=None):
    N = px.shape[-1]
    Q = qx.shape[-1]

    def one(qx1, qy1, qz1, px1, py1, pz1, f1):
        dx = qx1[:, None] - px1[None, :]
        dy = qy1[:, None] - py1[None, :]
        dz = qz1[:, None] - pz1[None, :]
        d2 = (dx * dx + dy * dy) + dz * dz
        mask = d2 <= r * r
        iota = jnp.arange(N, dtype=jnp.int32)
        selv = jnp.where(mask, -iota, -N - 1)
        vals, _ = lax.top_k(selv, KN)
        got = vals > -N - 1
        nbr = jnp.where(got, -vals, 0)
        cnt = jnp.sum(mask.astype(jnp.int32), axis=1)
        cnt = jnp.minimum(cnt, KN)
        relx = px1[nbr] - qx1[:, None]
        rely = py1[nbr] - qy1[:, None]
        relz = pz1[nbr] - qz1[:, None]
        rel = jnp.stack(
            [relx, rely, relz, jnp.zeros_like(relx)], axis=-1
        ).reshape(Q * KN, 4)
        fg = None if f1 is None else f1[nbr].reshape(Q * KN, -1)
        return rel, cnt.reshape(Q, 1, 1), fg

    return jax.vmap(one)(qx, qy, qz, px, py, pz, feats)


def _tw(w):
    return jnp.asarray(w.T, jnp.float32)


def _padt(w, k=4):
    wt = w.T
    return jnp.pad(wt, ((0, k - wt.shape[0]), (0, 0)))


def kernel(pos, params, batch):
    pr = pos.reshape(B, P, 3)
    px = pr[:, :, 0]
    py = pr[:, :, 1]
    pz = pr[:, :, 2]

    (q1x, q1y, q1z, q2x, q2y, q2z, q3x, q3y, q3z) = _run_fps(px, py, pz)

    p = params
    col = lambda ax, n: [a.reshape(B, n, 1) for a in ax]
    rowv = lambda ax, n: [a.reshape(B, 1, n) for a in ax]
    q1 = (q1x, q1y, q1z)
    q2 = (q2x, q2y, q2z)
    q3 = (q3x, q3y, q3z)
    pp = (px, py, pz)

    pk1 = _run_pack(col(q1, N1), rowv(pp, P), 0.2, N1, P, 256)
    pk2 = _run_pack(col(q2, N2), rowv(q1, N1), 0.4, N2, N1, 256)
    pk3 = _run_pack(col(q3, N3), rowv(q2, N2), 0.8, N3, N2, 64)

    rel1, nbr1, cnt1 = _make_compact(N1, P, 64)(pk1, *pp, *q1)
    rel2, nbr2, cnt2 = _make_compact(N2, N1, 64)(pk2, *q1, *q2)
    rel3, nbr3, cnt3 = _make_compact(N3, N2, 16)(pk3, *q2, *q3)

    x1 = _run_mlp(rel1, None, cnt1.reshape(B, N1, 1, 1), _padt(p["sa1_w1"]),
                  None, _tw(p["sa1_w2"]), p["sa1_b1"][None], p["sa1_b2"][None],
                  N1, 64, 128)

    xg2 = _make_rowgather(N2, 64, 16)(x1.reshape(B * N1, 64), nbr2)
    xg2 = xg2.reshape(B, N2 * KN, 64)
    x2 = _run_mlp(rel2, xg2, cnt2.reshape(B, N2, 1, 1),
                  _padt(p["sa2_w1"][:, 64:]),
                  _tw(p["sa2_w1"][:, :64]), _tw(p["sa2_w2"]),
                  p["sa2_b1"][None], p["sa2_b2"][None], N2, 128, 128)

    xg3 = _make_rowgather(N3, 128, 8)(x2.reshape(B * N2, 128), nbr3)
    xg3 = xg3.reshape(B, N3 * KN, 128)
    cnt3 = cnt3.reshape(B, N3, 1, 1)

    eps = jax.random.normal(jax.random.key(42), (B, 64), dtype=jnp.float32)
    eps = eps.reshape(B, 1, 64)
    xs = jnp.linspace(-0.3, 0.3, 50)
    gx, gy = jnp.meshgrid(xs, xs, indexing="ij")
    gridc = jnp.stack([gx.ravel(), gy.ravel()], axis=-1).astype(jnp.float32)

    posT = jnp.stack([px, py, pz], axis=1)

    wd = [
        _padt(p["sa3_w1"][:, 128:]), _tw(p["sa3_w1"][:, :128]),
        _tw(p["sa3_w2"]), p["sa3_b1"][None], p["sa3_b2"][None],
        _tw(p["mu_w"]), p["mu_b"][None], _tw(p["lv_w"]), p["lv_b"][None],
        _tw(p["f1_w1"][:, :64]), _tw(p["f1_w1"][:, 64:]), p["f1_b1"][None],
        _tw(p["f1_w2"]), p["f1_b2"][None], _tw(p["f1_w3"]), p["f1_b3"][None],
        _tw(p["f2_w1"][:, :64]), _tw(p["f2_w1"][:, 64:]), p["f2_b1"][None],
        _tw(p["f2_w2"]), p["f2_b2"][None], _tw(p["f2_w3"]), p["f2_b3"][None],
    ]
    mu, recon, loss, ch, kl = _run_final(xg3, rel3, cnt3, posT, eps, gridc, wd)
    return (loss.reshape(()), ch.reshape(()), kl.reshape(()),
            mu.reshape(B, 64), recon)
